# baseline (speedup 1.0000x reference)
import functools

import jax
import jax.numpy as jnp
from jax import lax
from jax.experimental import pallas as pl
from jax.experimental.pallas import tpu as pltpu
from jax.experimental.pallas import tpu_sc as plsc

VOCAB = 1000000
HV = VOCAB // 2
DIM = 64
WD = 128
B = 16384
K = 20

_info = plsc.get_sparse_core_info()
NC, NS = _info.num_cores, _info.num_subcores
NW = NC * NS
BPW = B // NW
G = 16
NG = BPW // G
CH = 80
NCH = (G * K) // CH


def _log_sigmoid_vec(x):
    u = jnp.exp(-jnp.abs(x))
    z = u / (2.0 + u)
    z2 = z * z
    p = 1.0 / 11.0
    p = p * z2 + 1.0 / 9.0
    p = p * z2 + 1.0 / 7.0
    p = p * z2 + 1.0 / 5.0
    p = p * z2 + 1.0 / 3.0
    p = p * z2 + 1.0
    return jnp.minimum(x, 0.0) - 2.0 * z * p


def _sc_body(tr, th, cr, ch_, nr, nh, emb, cemb, out,
             idx_tr, idx_th, idx_cr, idx_ch, idx_nr, idx_nh,
             tb, cb, nb, part_v, sem0, sem1):
    wid = lax.axis_index("s") * NC + lax.axis_index("c")
    base = wid * BPW
    sems = (sem0, sem1)

    pltpu.sync_copy(tr.at[pl.ds(base, BPW)], idx_tr)
    pltpu.sync_copy(th.at[pl.ds(base, BPW)], idx_th)
    pltpu.sync_copy(cr.at[pl.ds(base, BPW)], idx_cr)
    pltpu.sync_copy(ch_.at[pl.ds(base, BPW)], idx_ch)
    pltpu.sync_copy(nr.at[pl.ds(base * K, BPW * K)], idx_nr)
    pltpu.sync_copy(nh.at[pl.ds(base * K, BPW * K)], idx_nh)

    def descs(g, p):
        ds_ = [
            pltpu.make_async_copy(emb.at[idx_tr.at[pl.ds(g * G, G)]],
                                  tb.at[p], sems[p]),
            pltpu.make_async_copy(cemb.at[idx_cr.at[pl.ds(g * G, G)]],
                                  cb.at[p], sems[p]),
        ]
        for j in range(NCH):
            ds_.append(
                pltpu.make_async_copy(
                    cemb.at[idx_nr.at[pl.ds(g * G * K + j * CH, CH)]],
                    nb.at[p, pl.ds(j * CH, CH)], sems[p]))
        return ds_

    def fire(g, p):
        for d_ in descs(g, p):
            d_.start()

    def drain(g, p):
        for d_ in descs(g, p):
            d_.wait()

    def compute(g, p):
        tbp, cbp, nbp = tb.at[p], cb.at[p], nb.at[p]
        rows = lax.iota(jnp.int32, 16)
        nrows = [rows * K + k for k in range(K)]
        goff = g * G
        th16 = idx_th[pl.ds(goff, 16)]
        ch16 = idx_ch[pl.ds(goff, 16)]
        nh16 = [plsc.load_gather(idx_nh, [nrows[k] + g * (G * K)])
                for k in range(K)]

        def d_body(d, carry):
            ddrot = (jnp.full((16,), d, jnp.int32) + rows) & (DIM - 1)
            tvec = plsc.load_gather(tbp, [rows, th16 + ddrot])
            cvec = plsc.load_gather(cbp, [rows, ch16 + ddrot])
            out_ = [carry[0] + tvec * cvec]
            for k in range(K):
                nvec = plsc.load_gather(nbp, [nrows[k], nh16[k] + ddrot])
                out_.append(carry[1 + k] + tvec * nvec)
            return tuple(out_)

        init = tuple(jnp.zeros((16,), jnp.float32) for _ in range(K + 1))
        accs = lax.fori_loop(0, DIM, d_body, init)
        a = part_v[...] + _log_sigmoid_vec(accs[0])
        for j in range(1, K + 1):
            a = a + _log_sigmoid_vec(-accs[j])
        part_v[...] = a

    part_v[...] = jnp.zeros((16,), jnp.float32)
    fire(0, 0)
    fire(1, 1)

    def g_body(gg, _):
        g0 = 2 * gg
        drain(g0, 0)
        compute(g0, 0)
        fire(g0 + 2, 0)
        drain(g0 + 1, 1)
        compute(g0 + 1, 1)
        fire(g0 + 3, 1)
        return 0

    lax.fori_loop(0, NG // 2 - 1, g_body, 0)
    drain(NG - 2, 0)
    compute(NG - 2, 0)
    drain(NG - 1, 1)
    compute(NG - 1, 1)

    pltpu.sync_copy(part_v, out.at[pl.ds(wid * 16, 16)])


@functools.partial(
    pl.kernel,
    out_type=jax.ShapeDtypeStruct((NW * 16,), jnp.float32),
    mesh=plsc.VectorSubcoreMesh(core_axis_name="c", subcore_axis_name="s"),
    scratch_types=[
        pltpu.VMEM((BPW,), jnp.int32),
        pltpu.VMEM((BPW,), jnp.int32),
        pltpu.VMEM((BPW,), jnp.int32),
        pltpu.VMEM((BPW,), jnp.int32),
        pltpu.VMEM((BPW * K,), jnp.int32),
        pltpu.VMEM((BPW * K,), jnp.int32),
        pltpu.VMEM((2, G, WD), jnp.float32),
        pltpu.VMEM((2, G, WD), jnp.float32),
        pltpu.VMEM((2, G * K, WD), jnp.float32),
        pltpu.VMEM((16,), jnp.float32),
        pltpu.SemaphoreType.DMA,
        pltpu.SemaphoreType.DMA,
    ],
    compiler_params=pltpu.CompilerParams(use_tc_tiling_on_sc=True,
                                         needs_layout_passes=False),
)
def _sc_loss(tr, th, cr, ch_, nr, nh, emb, cemb, out,
             idx_tr, idx_th, idx_cr, idx_ch, idx_nr, idx_nh,
             tb, cb, nb, part_v, sem0, sem1):
    _sc_body(tr, th, cr, ch_, nr, nh, emb, cemb, out,
             idx_tr, idx_th, idx_cr, idx_ch, idx_nr, idx_nh,
             tb, cb, nb, part_v, sem0, sem1)


CBLK = 2048
RBLK = CBLK // 2
NBO = -(-VOCAB // CBLK)
HV2 = NBO * RBLK


def _tc_pair_xpose_kernel(a_ref, ao_ref):
    a = a_ref[...]
    ao_ref[...] = jnp.concatenate([a[:, :RBLK].T, a[:, RBLK:].T], axis=1)


def _tc_pair_xpose(embT):
    return pl.pallas_call(
        _tc_pair_xpose_kernel,
        grid=(NBO,),
        in_specs=[pl.BlockSpec((DIM, CBLK), lambda i: (0, i))],
        out_specs=pl.BlockSpec((RBLK, WD), lambda i: (i, 0)),
        out_shape=jax.ShapeDtypeStruct((HV2, WD), jnp.float32),
    )(embT)


CPB = RBLK // 128
UNITS = (NBO - 1) * CPB + 4
UPW = 124
TV0 = 999936


def _xp_body(src, tail, out, ba, bb, ob, tl, sem0, sem1):
    wid = lax.axis_index("s") * NC + lax.axis_index("c")
    sems = (sem0, sem1)

    def voffs(u):
        blk = u // CPB
        j = u - blk * CPB
        va = pl.multiple_of(blk * CBLK + j * 128, 128)
        return va, pl.multiple_of(va + RBLK, 128)

    def fire(u, p):
        va, vb = voffs(u)

        @pl.when(u < UNITS)
        def _():
            pltpu.make_async_copy(src.at[:, pl.ds(va, 128)], ba.at[p],
                                  sems[p]).start()

            @pl.when(vb + 128 <= VOCAB)
            def _():
                pltpu.make_async_copy(src.at[:, pl.ds(vb, 128)], bb.at[p],
                                      sems[p]).start()

    def drain(u, p):
        va, vb = voffs(u)

        @pl.when(u < UNITS)
        def _():
            pltpu.make_async_copy(src.at[:, pl.ds(va, 128)], ba.at[p],
                                  sems[p]).wait()

            @pl.when(vb + 128 <= VOCAB)
            def _():
                pltpu.make_async_copy(src.at[:, pl.ds(vb, 128)], bb.at[p],
                                      sems[p]).wait()

    def compute(u, p):
        va, _ = voffs(u)
        lanes16 = lax.iota(jnp.int32, 16)
        for vg in range(8):
            vl = vg * 16 + lanes16

            def d_body(d, _):
                dr = (jnp.full((16,), d, jnp.int32) + lanes16) & (DIM - 1)
                av = plsc.load_gather(ba.at[p], [dr, vl])
                plsc.store_scatter(ob, [vl, dr], av)
                bv = plsc.load_gather(bb.at[p], [dr, vl])
                plsc.store_scatter(ob, [vl, dr + DIM], bv)
                return 0

            lax.fori_loop(0, DIM, d_body, 0)
        orow = pl.multiple_of((va >> 11) * RBLK + (va & (RBLK - 1)), 128)
        pltpu.sync_copy(ob, out.at[pl.ds(orow, 128)])

    u0 = wid * UPW
    fire(u0, 0)
    fire(u0 + 1, 1)

    def step(u, p):
        drain(u, p)

        @pl.when(u < UNITS)
        def _():
            compute(u, p)

        fire(u + 2, p)

    def u_body(i, _):
        u = u0 + 2 * i
        step(u, 0)
        step(u + 1, 1)
        return 0

    lax.fori_loop(0, UPW // 2 - 1, u_body, 0)
    u_last = u0 + UPW - 2
    drain(u_last, 0)

    @pl.when(u_last < UNITS)
    def _():
        compute(u_last, 0)

    drain(u_last + 1, 1)

    @pl.when(u_last + 1 < UNITS)
    def _():
        compute(u_last + 1, 1)

    @pl.when(wid == NW - 1)
    def _():
        pltpu.sync_copy(tail, tl)
        lanes16 = lax.iota(jnp.int32, 16)
        for vg in range(4):
            vl = vg * 16 + lanes16

            def td_body(d, _):
                dr = (jnp.full((16,), d, jnp.int32) + lanes16) & (DIM - 1)
                tv = plsc.load_gather(tl, [dr * 64 + vl])
                plsc.store_scatter(ob, [vl, dr], tv)
                return 0

            lax.fori_loop(0, DIM, td_body, 0)
        orow = (TV0 >> 11) * RBLK + (TV0 & (RBLK - 1))
        pltpu.sync_copy(ob.at[pl.ds(0, 64)], out.at[pl.ds(orow, 64)])


@functools.partial(
    pl.kernel,
    out_type=jax.ShapeDtypeStruct((HV2, WD), jnp.float32),
    mesh=plsc.VectorSubcoreMesh(core_axis_name="c", subcore_axis_name="s"),
    scratch_types=[
        pltpu.VMEM((2, DIM, 128), jnp.float32),
        pltpu.VMEM((2, DIM, 128), jnp.float32),
        pltpu.VMEM((128, WD), jnp.float32),
        pltpu.VMEM((DIM * 64,), jnp.float32),
        pltpu.SemaphoreType.DMA,
        pltpu.SemaphoreType.DMA,
    ],
    compiler_params=pltpu.CompilerParams(use_tc_tiling_on_sc=True,
                                         needs_layout_passes=False),
)
def sc_pair_xpose(src, tail, out, ba, bb, ob, tl, sem0, sem1):
    _xp_body(src, tail, out, ba, bb, ob, tl, sem0, sem1)


def kernel(target, context, negative_samples, embeddings, context_embeddings):
    tgt = target.astype(jnp.int32)
    ctx = context.astype(jnp.int32)
    negf = negative_samples.astype(jnp.int32).reshape(-1)

    emb2 = _tc_pair_xpose(embeddings.T)
    cemb2 = sc_pair_xpose(context_embeddings.T,
                          context_embeddings.T[:, TV0:].reshape(-1))

    def prow(v):
        return ((v >> 11) << 10) + (v & 1023)

    def phalf(v):
        return ((v >> 10) & 1) * DIM

    parts = _sc_loss(prow(tgt), phalf(tgt), prow(ctx), phalf(ctx),
                     prow(negf), phalf(negf), emb2, cemb2)
    return -jnp.sum(parts) / B
